# baseline (speedup 1.0000x reference)
import functools

import jax
import jax.numpy as jnp
from jax import lax
from jax.experimental import pallas as pl
from jax.experimental.pallas import tpu as pltpu
from jax.experimental.pallas import tpu_sc as plsc

N = 10000
E = 10000
NNZ = 320000
NSUB = 16
DUMMY = N
ACC_ROWS = 10112
BN = 1000
NB = N // BN


_EXP_NO_SCATTER = True
_EXP_GW = 128


def _make_sc_segsum(kj: int, gw: int = 64):
    assert kj % 2 == 0
    zslc = ACC_ROWS // NSUB

    mesh = plsc.VectorSubcoreMesh(core_axis_name="c", subcore_axis_name="s")

    @functools.partial(
        pl.kernel,
        out_type=jax.ShapeDtypeStruct((2, 2, ACC_ROWS, 64), jnp.float32),
        mesh=mesh,
        compiler_params=pltpu.CompilerParams(use_tc_tiling_on_sc=False),
        scratch_types=[
            pltpu.VMEM((kj, 128), jnp.int32),
            pltpu.VMEM((kj, 128), jnp.int32),
            pltpu.VMEM((128, gw), jnp.float32),
            pltpu.VMEM((128, gw), jnp.float32),
            pltpu.VMEM_SHARED((ACC_ROWS, 64), jnp.float32),
            pltpu.SemaphoreType.DMA,
            pltpu.SemaphoreType.DMA,
            pltpu.SemaphoreType.DMA,
            pltpu.SemaphoreType.DMA,
        ],
    )
    def segsum(table_hbm, gidx_hbm, sidx_hbm, zeros_hbm, out_hbm,
               gi_v, si_v, r0, r1, acc, g0, g1, s0, s1):
        c = lax.axis_index("c")
        w = lax.axis_index("s")
        pltpu.sync_copy(sidx_hbm.at[c, w], si_v)

        def gs(j, buf, sem):
            pltpu.async_copy(table_hbm.at[gi_v.at[j]], buf, sem)

        def gw(j, buf, sem):
            pltpu.make_async_copy(table_hbm.at[gi_v.at[j]], buf, sem).wait()

        def ss(j, buf, sem):
            if not _EXP_NO_SCATTER:
                pltpu.async_copy(buf, acc.at[si_v.at[j]], sem, add=True)

        def sw(j, buf, sem):
            if not _EXP_NO_SCATTER:
                pltpu.make_async_copy(buf, acc.at[si_v.at[j]], sem).wait()

        for h in range(2):
            pltpu.sync_copy(zeros_hbm.at[pl.ds(w * zslc, zslc)],
                            acc.at[pl.ds(w * zslc, zslc)])
            pltpu.sync_copy(gidx_hbm.at[c, h, w], gi_v)
            plsc.subcore_barrier()

            gs(0, r0, g0)

            def body(i, carry):
                j0 = 2 * i
                j1 = j0 + 1
                gw(j0, r0, g0)
                ss(j0, r0, s0)

                @pl.when(i >= 1)
                def _():
                    sw(j0 - 1, r1, s1)

                gs(j1, r1, g1)
                gw(j1, r1, g1)
                ss(j1, r1, s1)
                sw(j0, r0, s0)

                @pl.when(i + 1 < kj // 2)
                def _():
                    gs(j0 + 2, r0, g0)

                return carry

            lax.fori_loop(0, kj // 2, body, 0)
            sw(kj - 1, r1, s1)
            plsc.subcore_barrier()
            pltpu.sync_copy(acc.at[pl.ds(w * zslc, zslc)],
                            out_hbm.at[c, h, pl.ds(w * zslc, zslc)])

    return segsum


_sc_segsum_full = _make_sc_segsum(158, _EXP_GW)
_sc_segsum_half = _make_sc_segsum(80, _EXP_GW)


def _pad_idx(idx, n_sc, fill):
    per = idx.shape[0] // (n_sc * NSUB)
    kj = -(-per // 128)
    kj += kj % 2
    a = idx.reshape(n_sc, NSUB, per)
    a = jnp.pad(a, ((0, 0), (0, 0), (0, kj * 128 - per)), constant_values=fill)
    return a.reshape(n_sc, NSUB, kj, 128)


def _quarter_gidx(base, rows):
    return jnp.stack([
        jnp.stack([2 * (c * rows + base) + h for h in range(2)])
        for c in range(2)
    ])


def _vspec():
    return pl.BlockSpec((BN, 1), lambda i: (i, 0))


def _qspec():
    return pl.BlockSpec((2, 2, BN, 64), lambda i: (0, 0, i, 0))


def _split_spec():
    return pl.BlockSpec((2, BN, 128), lambda i: (0, i, 0))


def _cat(q_ref, c):
    return jnp.concatenate([q_ref[c, 0], q_ref[c, 1]], axis=1)


def _tc1_body(x_ref, w_ref, b_ref, dvb_ref, out_ref):
    h = jnp.dot(x_ref[...], w_ref[...], preferred_element_type=jnp.float32)
    h = (h + b_ref[...]) * dvb_ref[...]
    out_ref[0] = h[:, :128]
    out_ref[1] = h[:, 128:]


_tc1 = pl.pallas_call(
    _tc1_body,
    grid=(NB,),
    in_specs=[
        pl.BlockSpec((BN, 128), lambda i: (i, 0)),
        pl.BlockSpec((128, 256), lambda i: (0, 0)),
        pl.BlockSpec((1, 256), lambda i: (0, 0)),
        _vspec(),
    ],
    out_specs=_split_spec(),
    out_shape=jax.ShapeDtypeStruct((2, N, 128), jnp.float32),
)


def _tc2_body(a_ref, debi_ref, dea_ref, u_ref, c_ref, out_ref):
    debi = debi_ref[...]
    t0 = jax.nn.relu(_cat(a_ref, 0) * debi)
    t1 = jax.nn.relu(_cat(a_ref, 1) * debi)
    o = jnp.dot(t0, u_ref[:128, :], preferred_element_type=jnp.float32)
    o += jnp.dot(t1, u_ref[128:, :], preferred_element_type=jnp.float32)
    o = (o + c_ref[...]) * dea_ref[...]
    out_ref[0] = o[:, :128]
    out_ref[1] = o[:, 128:]


_tc2 = pl.pallas_call(
    _tc2_body,
    grid=(NB,),
    in_specs=[
        _qspec(),
        _vspec(),
        _vspec(),
        pl.BlockSpec((256, 256), lambda i: (0, 0)),
        pl.BlockSpec((1, 256), lambda i: (0, 0)),
    ],
    out_specs=_split_spec(),
    out_shape=jax.ShapeDtypeStruct((2, E, 128), jnp.float32),
)


def _tc3_body(a_ref, dvai_ref, dvb_ref, w_ref, b_ref, out_ref):
    dvai = dvai_ref[...]
    t0 = jax.nn.relu(_cat(a_ref, 0) * dvai)
    t1 = jax.nn.relu(_cat(a_ref, 1) * dvai)
    h = jnp.dot(t0, w_ref[:128, :], preferred_element_type=jnp.float32)
    h += jnp.dot(t1, w_ref[128:, :], preferred_element_type=jnp.float32)
    h = (h + b_ref[...]) * dvb_ref[...]
    out_ref[0] = h[:, :128]
    out_ref[1] = h[:, 128:]


_tc3 = pl.pallas_call(
    _tc3_body,
    grid=(NB,),
    in_specs=[
        _qspec(),
        _vspec(),
        _vspec(),
        pl.BlockSpec((256, 256), lambda i: (0, 0)),
        pl.BlockSpec((1, 256), lambda i: (0, 0)),
    ],
    out_specs=_split_spec(),
    out_shape=jax.ShapeDtypeStruct((2, N, 128), jnp.float32),
)


def _tc4_body(a_ref, debi_ref, dea_ref, u_ref, c_ref, e_ref, o_ref):
    debi = debi_ref[...]
    e0 = _cat(a_ref, 0) * debi
    e1 = _cat(a_ref, 1) * debi
    e_ref[:, :128] = e0
    e_ref[:, 128:] = e1
    o = jnp.dot(jax.nn.relu(e0), u_ref[:128, :],
                preferred_element_type=jnp.float32)
    o += jnp.dot(jax.nn.relu(e1), u_ref[128:, :],
                 preferred_element_type=jnp.float32)
    o_ref[...] = (o + c_ref[...]) * dea_ref[...]


_tc4 = pl.pallas_call(
    _tc4_body,
    grid=(NB,),
    in_specs=[
        _qspec(),
        _vspec(),
        _vspec(),
        pl.BlockSpec((256, 128), lambda i: (0, 0)),
        pl.BlockSpec((1, 128), lambda i: (0, 0)),
    ],
    out_specs=[
        pl.BlockSpec((BN, 256), lambda i: (i, 0)),
        pl.BlockSpec((BN, 128), lambda i: (i, 0)),
    ],
    out_shape=[
        jax.ShapeDtypeStruct((E, 256), jnp.float32),
        jax.ShapeDtypeStruct((E, 128), jnp.float32),
    ],
)


def _tc5_body(p_ref, dvai_ref, out_ref):
    lo = p_ref[0, 0] + p_ref[1, 0]
    hi = p_ref[0, 1] + p_ref[1, 1]
    out_ref[...] = jnp.concatenate([lo, hi], axis=1) * dvai_ref[...]


_tc5 = pl.pallas_call(
    _tc5_body,
    grid=(NB,),
    in_specs=[_qspec(), _vspec()],
    out_specs=pl.BlockSpec((BN, 128), lambda i: (i, 0)),
    out_shape=jax.ShapeDtypeStruct((N, 128), jnp.float32),
)


def kernel(x, hyperedge_index, D_v_beta, D_e_beta_inv, D_e_alpha, D_v_alpha_inv,
           W1, b1, U1, c1, W2, b2, U2, c2):
    src = hyperedge_index[0]
    dst = hyperedge_index[1]

    src_g = _pad_idx(src, 1, 0)[0]
    src_s = _pad_idx(src, 1, DUMMY)[0]
    dst_g = _pad_idx(dst, 1, 0)[0]
    dst_s = _pad_idx(dst, 1, DUMMY)[0]
    gidx_a = _quarter_gidx(src_g, N)
    sidx_a = jnp.stack([dst_s, dst_s])
    gidx_b = _quarter_gidx(dst_g, E)
    sidx_b = jnp.stack([src_s, src_s])
    dst_g2 = _pad_idx(dst, 2, 0)
    gidx_d = jnp.stack([2 * dst_g2[c] + jnp.arange(2).reshape(2, 1, 1, 1)
                        for c in range(2)])
    sidx_d = _pad_idx(src, 2, DUMMY)

    zeros = jnp.zeros((ACC_ROWS, 64), jnp.float32)
    dvb = D_v_beta.reshape(N, 1)
    debi = D_e_beta_inv.reshape(E, 1)
    dea = D_e_alpha.reshape(E, 1)
    dvai = D_v_alpha_inv.reshape(N, 1)

    if _EXP_GW == 128:
        tshape = lambda t, r: t.reshape(2 * r, 128)
        gidx_a, gidx_b = _quarter_gidx(src_g, N) // 2, _quarter_gidx(dst_g, E) // 2
        gidx_d = gidx_d // 2
    elif _EXP_GW == 32:
        tshape = lambda t, r: t.reshape(8 * r, 32)
        gidx_a, gidx_b, gidx_d = 2 * gidx_a, 2 * gidx_b, 2 * gidx_d
    else:
        tshape = lambda t, r: t.reshape(4 * r, 64)

    h = _tc1(x, W1, b1.reshape(1, 256), dvb)
    ae = _sc_segsum_full(tshape(h, N), gidx_a, sidx_a, zeros)
    o = _tc2(ae, debi, dea, U1, c1.reshape(1, 256))
    av = _sc_segsum_full(tshape(o, E), gidx_b, sidx_b, zeros)
    h2 = _tc3(av, dvai, dvb, W2, b2.reshape(1, 256))
    ae2 = _sc_segsum_full(tshape(h2, N), gidx_a, sidx_a, zeros)
    e_out, o2 = _tc4(ae2, debi, dea, U2, c2.reshape(1, 128))
    p = _sc_segsum_half(tshape(o2, E // 2), gidx_d, sidx_d, zeros)
    out = _tc5(p, dvai)
    return (out, e_out)


# speedup vs baseline: 1.4964x; 1.4964x over previous
import functools

import jax
import jax.numpy as jnp
import numpy as np
from jax import lax
from jax.experimental import pallas as pl
from jax.experimental.pallas import tpu as pltpu
from jax.experimental.pallas import tpu_sc as plsc

N = 10000
E = 10000
NNZ = 320000
NSUB = 16
DUMMY = N
ACC_ROWS = 10112
ZSLC = ACC_ROWS // NSUB
BN = 1000
NB = N // BN

def _pack_half(hf):
    b = jax.lax.bitcast_convert_type(hf, jnp.int32) + jnp.int32(0x8000)
    lo = (b[:, :64] >> 16) & jnp.int32(0xFFFF)
    hi = b[:, 64:] & jnp.int32(-65536)
    return lo | hi


def _make_sc_segsum(kj: int):
    assert kj % 4 == 0
    mesh = plsc.VectorSubcoreMesh(core_axis_name="c", subcore_axis_name="s")

    @functools.partial(
        pl.kernel,
        out_type=jax.ShapeDtypeStruct((2, ACC_ROWS, 128), jnp.float32),
        mesh=mesh,
        compiler_params=pltpu.CompilerParams(use_tc_tiling_on_sc=False),
        scratch_types=[
            pltpu.VMEM((128,), jnp.int32),
            pltpu.VMEM((128,), jnp.int32),
            pltpu.VMEM((128,), jnp.int32),
            pltpu.VMEM((128,), jnp.int32),
            pltpu.VMEM((128,), jnp.int32),
            pltpu.VMEM((128,), jnp.int32),
            pltpu.VMEM((128, 64), jnp.int32),
            pltpu.VMEM((128, 64), jnp.int32),
            pltpu.VMEM((128, 128), jnp.float32),
            pltpu.VMEM((128, 128), jnp.float32),
            pltpu.VMEM_SHARED((ACC_ROWS, 128), jnp.float32),
            pltpu.SemaphoreType.DMA,
            pltpu.SemaphoreType.DMA,
            pltpu.SemaphoreType.DMA,
            pltpu.SemaphoreType.DMA,
            pltpu.SemaphoreType.DMA,
            pltpu.SemaphoreType.DMA,
        ],
    )
    def segsum(table_hbm, cidx_hbm, zeros_hbm, out_hbm,
               gi0, gi1, si0, si1, si2, si3, gb0, gb1, fb0, fb1, acc,
               c0, c1, g0, g1, s0, s1):
        c = lax.axis_index("c")
        w = lax.axis_index("s")
        GI, SI, GB, FB = (gi0, gi1), (si0, si1, si2, si3), (gb0, gb1), (fb0, fb1)
        CS, GS_, SS_ = (c0, c1), (g0, g1), (s0, s1)

        def il(j, b2):
            pltpu.async_copy(cidx_hbm.at[c, w, j], GI[b2], CS[b2])

        def iw(j, b2, b4):
            pltpu.make_async_copy(cidx_hbm.at[c, w, j], GI[b2], CS[b2]).wait()
            for g in range(8):
                x = GI[b2][pl.ds(16 * g, 16)]
                SI[b4][pl.ds(16 * g, 16)] = x >> 16
                GI[b2][pl.ds(16 * g, 16)] = x & jnp.int32(0xFFFF)

        def gs(b2):
            pltpu.async_copy(table_hbm.at[GI[b2]], GB[b2], GS_[b2])

        def gw(b2):
            pltpu.make_async_copy(table_hbm.at[GI[b2]], GB[b2], GS_[b2]).wait()

        def ss(b2, b4):
            pltpu.async_copy(FB[b2], acc.at[SI[b4]], SS_[b2], add=True)

        def sw(b2, b4):
            pltpu.make_async_copy(FB[b2], acc.at[SI[b4]], SS_[b2]).wait()

        def convert(gb, fb):
            def crow(r, carry):
                for g in range(4):
                    x = gb[r, pl.ds(16 * g, 16)]
                    lo = lax.bitcast_convert_type(x << 16, jnp.float32)
                    hi = lax.bitcast_convert_type(x & jnp.int32(-65536),
                                                  jnp.float32)
                    fb[r, pl.ds(16 * g, 16)] = lo
                    fb[r, pl.ds(64 + 16 * g, 16)] = hi
                return carry
            lax.fori_loop(0, 128, crow, 0)

        pltpu.sync_copy(zeros_hbm.at[pl.ds(w * ZSLC, ZSLC)],
                        acc.at[pl.ds(w * ZSLC, ZSLC)])
        plsc.subcore_barrier()

        il(0, 0)
        iw(0, 0, 0)
        gs(0)
        il(1, 1)
        iw(1, 1, 1)
        gs(1)

        def body(i, carry):
            j = 4 * i
            for k in range(4):
                b2 = k % 2
                gw(b2)

                if k < 2:
                    @pl.when(i >= 1)
                    def _():
                        sw(b2, (k + 2) % 4)
                else:
                    sw(b2, (k + 2) % 4)

                @pl.when(j + k + 2 < kj)
                def _():
                    il(j + k + 2, b2)

                convert(GB[b2], FB[b2])
                ss(b2, k)

                @pl.when(j + k + 2 < kj)
                def _():
                    iw(j + k + 2, b2, (k + 2) % 4)
                    gs(b2)
            return carry

        lax.fori_loop(0, kj // 4, body, 0)
        sw(0, 2)
        sw(1, 3)
        plsc.subcore_barrier()
        pltpu.sync_copy(acc.at[pl.ds(w * ZSLC, ZSLC)],
                        out_hbm.at[c, pl.ds(w * ZSLC, ZSLC)])

    return segsum


_sc_segsum_full = _make_sc_segsum(160)
_sc_segsum_half = _make_sc_segsum(80)


def _pad_idx(idx, n_sc, fill):
    per = idx.shape[0] // (n_sc * NSUB)
    kj = -(-per // 128)
    kj += (-kj) % 4
    a = idx.reshape(n_sc, NSUB, per)
    a = jnp.pad(a, ((0, 0), (0, 0), (0, kj * 128 - per)), constant_values=fill)
    return a.reshape(n_sc, NSUB, kj, 128)


def _vspec():
    return pl.BlockSpec((BN, 1), lambda i: (i, 0))


def _split_spec():
    return pl.BlockSpec((2, BN, 128), lambda i: (0, i, 0))


def _packed_spec():
    return pl.BlockSpec((2, BN, 64), lambda i: (0, i, 0))


def _tc1_body(x_ref, w_ref, b_ref, dvb_ref, out_ref):
    h = jnp.dot(x_ref[...], w_ref[...], preferred_element_type=jnp.float32)
    h = (h + b_ref[...]) * dvb_ref[...]
    out_ref[0] = _pack_half(h[:, :128])
    out_ref[1] = _pack_half(h[:, 128:])


_tc1 = pl.pallas_call(
    _tc1_body,
    grid=(NB,),
    in_specs=[
        pl.BlockSpec((BN, 128), lambda i: (i, 0)),
        pl.BlockSpec((128, 256), lambda i: (0, 0)),
        pl.BlockSpec((1, 256), lambda i: (0, 0)),
        _vspec(),
    ],
    out_specs=_packed_spec(),
    out_shape=jax.ShapeDtypeStruct((2, N, 64), jnp.int32),
)


def _tc2_body(a_ref, debi_ref, dea_ref, u_ref, c_ref, out_ref):
    debi = debi_ref[...]
    t0 = jax.nn.relu(a_ref[0] * debi)
    t1 = jax.nn.relu(a_ref[1] * debi)
    o = jnp.dot(t0, u_ref[:128, :], preferred_element_type=jnp.float32)
    o += jnp.dot(t1, u_ref[128:, :], preferred_element_type=jnp.float32)
    o = (o + c_ref[...]) * dea_ref[...]
    out_ref[0] = _pack_half(o[:, :128])
    out_ref[1] = _pack_half(o[:, 128:])


_tc2 = pl.pallas_call(
    _tc2_body,
    grid=(NB,),
    in_specs=[
        _split_spec(),
        _vspec(),
        _vspec(),
        pl.BlockSpec((256, 256), lambda i: (0, 0)),
        pl.BlockSpec((1, 256), lambda i: (0, 0)),
    ],
    out_specs=_packed_spec(),
    out_shape=jax.ShapeDtypeStruct((2, E, 64), jnp.int32),
)


def _tc3_body(a_ref, dvai_ref, dvb_ref, w_ref, b_ref, out_ref):
    dvai = dvai_ref[...]
    t0 = jax.nn.relu(a_ref[0] * dvai)
    t1 = jax.nn.relu(a_ref[1] * dvai)
    h = jnp.dot(t0, w_ref[:128, :], preferred_element_type=jnp.float32)
    h += jnp.dot(t1, w_ref[128:, :], preferred_element_type=jnp.float32)
    h = (h + b_ref[...]) * dvb_ref[...]
    out_ref[0] = _pack_half(h[:, :128])
    out_ref[1] = _pack_half(h[:, 128:])


_tc3 = pl.pallas_call(
    _tc3_body,
    grid=(NB,),
    in_specs=[
        _split_spec(),
        _vspec(),
        _vspec(),
        pl.BlockSpec((256, 256), lambda i: (0, 0)),
        pl.BlockSpec((1, 256), lambda i: (0, 0)),
    ],
    out_specs=_packed_spec(),
    out_shape=jax.ShapeDtypeStruct((2, N, 64), jnp.int32),
)


def _tc4_body(a_ref, debi_ref, dea_ref, u_ref, c_ref, e_ref, o_ref):
    debi = debi_ref[...]
    e0 = a_ref[0] * debi
    e1 = a_ref[1] * debi
    e_ref[:, :128] = e0
    e_ref[:, 128:] = e1
    o = jnp.dot(jax.nn.relu(e0), u_ref[:128, :],
                preferred_element_type=jnp.float32)
    o += jnp.dot(jax.nn.relu(e1), u_ref[128:, :],
                 preferred_element_type=jnp.float32)
    o = (o + c_ref[...]) * dea_ref[...]
    o_ref[...] = _pack_half(o)


_tc4 = pl.pallas_call(
    _tc4_body,
    grid=(NB,),
    in_specs=[
        _split_spec(),
        _vspec(),
        _vspec(),
        pl.BlockSpec((256, 128), lambda i: (0, 0)),
        pl.BlockSpec((1, 128), lambda i: (0, 0)),
    ],
    out_specs=[
        pl.BlockSpec((BN, 256), lambda i: (i, 0)),
        pl.BlockSpec((BN, 64), lambda i: (i, 0)),
    ],
    out_shape=[
        jax.ShapeDtypeStruct((E, 256), jnp.float32),
        jax.ShapeDtypeStruct((E, 64), jnp.int32),
    ],
)


def _tc5_body(p_ref, dvai_ref, out_ref):
    out_ref[...] = (p_ref[0] + p_ref[1]) * dvai_ref[...]


_tc5 = pl.pallas_call(
    _tc5_body,
    grid=(NB,),
    in_specs=[_split_spec(), _vspec()],
    out_specs=pl.BlockSpec((BN, 128), lambda i: (i, 0)),
    out_shape=jax.ShapeDtypeStruct((N, 128), jnp.float32),
)


def kernel(x, hyperedge_index, D_v_beta, D_e_beta_inv, D_e_alpha, D_v_alpha_inv,
           W1, b1, U1, c1, W2, b2, U2, c2):
    src = hyperedge_index[0]
    dst = hyperedge_index[1]

    src_g = _pad_idx(src, 1, 0)[0]
    src_s = _pad_idx(src, 1, DUMMY)[0]
    dst_g = _pad_idx(dst, 1, 0)[0]
    dst_s = _pad_idx(dst, 1, DUMMY)[0]
    dst_sh = dst_s << 16
    src_sh = src_s << 16
    cidx_a = jnp.stack([src_g | dst_sh, (src_g + N) | dst_sh])
    cidx_b = jnp.stack([dst_g | src_sh, (dst_g + E) | src_sh])
    cidx_d = _pad_idx(dst, 2, 0) | (_pad_idx(src, 2, DUMMY) << 16)

    zeros = jnp.zeros((ACC_ROWS, 128), jnp.float32)
    dvb = D_v_beta.reshape(N, 1)
    debi = D_e_beta_inv.reshape(E, 1)
    dea = D_e_alpha.reshape(E, 1)
    dvai = D_v_alpha_inv.reshape(N, 1)

    h = _tc1(x, W1, b1.reshape(1, 256), dvb)
    ae = _sc_segsum_full(h.reshape(2 * N, 64), cidx_a, zeros)
    o = _tc2(ae, debi, dea, U1, c1.reshape(1, 256))
    av = _sc_segsum_full(o.reshape(2 * E, 64), cidx_b, zeros)
    h2 = _tc3(av, dvai, dvb, W2, b2.reshape(1, 256))
    ae2 = _sc_segsum_full(h2.reshape(2 * N, 64), cidx_a, zeros)
    e_out, o2 = _tc4(ae2, debi, dea, U2, c2.reshape(1, 128))
    p = _sc_segsum_half(o2, cidx_d, zeros)
    out = _tc5(p, dvai)
    return (out, e_out)
